# baseline (speedup 1.0000x reference)
import jax
import jax.numpy as jnp
from jax import lax
from jax.experimental import pallas as pl
from jax.experimental.pallas import tpu as pltpu
from jax.experimental.pallas import tpu_sc as plsc

N = 10000
E = 320000
D = 128

NC = 2
NS = 16
NP = 10240
ROWS_PER_TILE = NP // NS
E_PER_TILE = E // (NC * NS)
CH = 80
NCHUNK = E_PER_TILE // CH
ZROWS = 40
NBUF = 3

_MESH = plsc.VectorSubcoreMesh(core_axis_name="c", subcore_axis_name="s")


def _propagate_body(xs_hbm, src_hbm, dst_hbm, agg_hbm, sidx_all, rows, dbuf,
                    zbuf, acc, sgs, sss, sds):
    c = lax.axis_index("c")
    s = lax.axis_index("s")

    zero16 = jnp.zeros((16,), jnp.float32)

    def fz(i, carry):
        for j in range(D // 16):
            zbuf[i, pl.ds(j * 16, 16)] = zero16
        return carry

    lax.fori_loop(0, ZROWS, fz, 0)

    def fcz(j, carry):
        pltpu.async_copy(
            zbuf, acc.at[pl.ds(s * ROWS_PER_TILE + j * ZROWS, ZROWS)], sss[0]
        )
        return carry

    lax.fori_loop(0, ROWS_PER_TILE // ZROWS, fcz, 0)

    def fcw(j, carry):
        pltpu.make_async_copy(
            zbuf, acc.at[pl.ds(s * ROWS_PER_TILE + j * ZROWS, ZROWS)], sss[0]
        ).wait()
        return carry

    lax.fori_loop(0, ROWS_PER_TILE // ZROWS, fcw, 0)

    tb = (c * NS + s) * E_PER_TILE
    pltpu.sync_copy(src_hbm.at[pl.ds(tb, E_PER_TILE)], sidx_all)
    plsc.subcore_barrier()

    def _prefetch(i, p):
        pltpu.async_copy(dst_hbm.at[pl.ds(tb + i * CH, CH)], dbuf[p], sds[p])
        pltpu.async_copy(
            xs_hbm.at[sidx_all.at[pl.ds(i * CH, CH)]], rows[p], sgs[p]
        )

    def _wait_pref(i, p):
        pltpu.make_async_copy(dst_hbm.at[pl.ds(tb, CH)], dbuf[p],
                              sds[p]).wait()
        pltpu.make_async_copy(
            xs_hbm.at[sidx_all.at[pl.ds(i * CH, CH)]], rows[p], sgs[p]
        ).wait()

    def _scatter(p):
        pltpu.async_copy(rows[p], acc.at[dbuf[p]], sss[p], add=True)

    def _wait_scatter(p):
        pltpu.make_async_copy(rows[p], acc.at[dbuf[p]], sss[p]).wait()

    for p in range(NBUF):
        _prefetch(p, p)

    NK = NCHUNK // NBUF

    def body(k, carry):
        i0 = k * NBUF
        for p in range(NBUF):
            _wait_pref(i0 + p, p)
            _scatter(p)

        @pl.when(k < NK - 1)
        def _():
            for p in range(NBUF):
                _wait_scatter(p)
                _prefetch(i0 + NBUF + p, p)

        return carry

    lax.fori_loop(0, NK, body, 0)
    for p in range(NBUF):
        _wait_scatter(p)

    for r in range(NCHUNK - NK * NBUF):
        i = NK * NBUF + r
        _prefetch(i, 0)
        _wait_pref(i, 0)
        _scatter(0)
        _wait_scatter(0)

    plsc.subcore_barrier()
    pltpu.sync_copy(
        acc.at[pl.ds(s * ROWS_PER_TILE, ROWS_PER_TILE)],
        agg_hbm.at[c].at[pl.ds(s * ROWS_PER_TILE, ROWS_PER_TILE)],
    )


_propagate_call = pl.kernel(
    _propagate_body,
    out_type=jax.ShapeDtypeStruct((NC, NP, D), jnp.float32),
    mesh=_MESH,
    scratch_types=[
        pltpu.VMEM((E_PER_TILE,), jnp.int32),
        [pltpu.VMEM((CH, D), jnp.float32) for _ in range(NBUF)],
        [pltpu.VMEM((CH,), jnp.int32) for _ in range(NBUF)],
        pltpu.VMEM((ZROWS, D), jnp.float32),
        pltpu.VMEM_SHARED((NP, D), jnp.float32),
        [pltpu.SemaphoreType.DMA for _ in range(NBUF)],
        [pltpu.SemaphoreType.DMA for _ in range(NBUF)],
        [pltpu.SemaphoreType.DMA for _ in range(NBUF)],
    ],
)


BN = 400


def _scales(dgo_blk, dgi_blk):
    deg_out = dgo_blk[0, :, 0:1] + dgo_blk[1, :, 0:1]
    deg_in = dgi_blk[0, :, 0:1] + dgi_blk[1, :, 0:1]
    s_out = lax.rsqrt(jnp.maximum(deg_out, 1.0))
    s_in = lax.rsqrt(jnp.maximum(deg_in, 1.0))
    return s_out, s_in


_deg_spec = pl.BlockSpec((NC, BN, 16), lambda i: (0, i, 0))


def _pre_body(x_ref, dgo_ref, dgi_ref, out_ref):
    s_out, _ = _scales(dgo_ref[...], dgi_ref[...])
    out_ref[...] = x_ref[...] * s_out


_pre_call = pl.pallas_call(
    _pre_body,
    grid=(N // BN,),
    in_specs=[
        pl.BlockSpec((BN, D), lambda i: (i, 0)),
        _deg_spec,
        _deg_spec,
    ],
    out_specs=pl.BlockSpec((BN, D), lambda i: (i, 0)),
    out_shape=jax.ShapeDtypeStruct((N, D), jnp.float32),
)


def _mid_body(agg_ref, dgo_ref, dgi_ref, w_ref, b_ref, out_ref):
    s_out, s_in = _scales(dgo_ref[...], dgi_ref[...])
    a = agg_ref[0, :, :] + agg_ref[1, :, :]
    y = jnp.dot(a * s_in, w_ref[...], preferred_element_type=jnp.float32)
    out_ref[...] = (y + b_ref[...]) * s_out


_mid_call = pl.pallas_call(
    _mid_body,
    grid=(N // BN,),
    in_specs=[
        pl.BlockSpec((NC, BN, D), lambda i: (0, i, 0)),
        _deg_spec,
        _deg_spec,
        pl.BlockSpec((D, D), lambda i: (0, 0)),
        pl.BlockSpec((1, D), lambda i: (0, 0)),
    ],
    out_specs=pl.BlockSpec((BN, D), lambda i: (i, 0)),
    out_shape=jax.ShapeDtypeStruct((N, D), jnp.float32),
)


def _final_body(agg_ref, dgo_ref, dgi_ref, w4_ref, b4_ref, wl_ref, bl_ref,
                out_ref):
    _, s_in = _scales(dgo_ref[...], dgi_ref[...])
    a = agg_ref[0, :, :] + agg_ref[1, :, :]
    y = jnp.dot(a * s_in, w4_ref[...], preferred_element_type=jnp.float32)
    y = y + b4_ref[...]
    z = jnp.dot(y, wl_ref[...], preferred_element_type=jnp.float32)
    out_ref[...] = jnp.maximum(z + bl_ref[...], 0.0)


_final_call = pl.pallas_call(
    _final_body,
    grid=(N // BN,),
    in_specs=[
        pl.BlockSpec((NC, BN, D), lambda i: (0, i, 0)),
        _deg_spec,
        _deg_spec,
        pl.BlockSpec((D, D), lambda i: (0, 0)),
        pl.BlockSpec((1, D), lambda i: (0, 0)),
        pl.BlockSpec((D, D), lambda i: (0, 0)),
        pl.BlockSpec((1, D), lambda i: (0, 0)),
    ],
    out_specs=pl.BlockSpec((BN, D), lambda i: (i, 0)),
    out_shape=jax.ShapeDtypeStruct((N, D), jnp.float32),
)


def kernel(in_feat, edge_index, W0, b0, W1, b1, W2, b2, W3, b3, W4, b4, Wl, bl):
    src = edge_index[0]
    dst = edge_index[1]
    ones_mat = jnp.ones((N, D), jnp.float32)

    zeros_e = jnp.zeros((E,), jnp.int32)
    dgo = _propagate_call(ones_mat, zeros_e, src)[:, :, :16]
    dgi = _propagate_call(ones_mat + jnp.minimum(dgo[0, 0, 0], 0.0), zeros_e,
                          dst)[:, :, :16]

    xs = _pre_call(in_feat, dgo, dgi)
    for W, b in zip([W0, W1, W2, W3], [b0, b1, b2, b3]):
        agg = _propagate_call(xs, src, dst)
        xs = _mid_call(agg, dgo, dgi, W, b.reshape(1, D))
    agg = _propagate_call(xs, src, dst)
    return _final_call(agg, dgo, dgi, W4, b4.reshape(1, D), Wl, bl.reshape(1, D))


# speedup vs baseline: 24.8783x; 24.8783x over previous
import jax
import jax.numpy as jnp
from jax import lax
from jax.experimental import pallas as pl
from jax.experimental.pallas import tpu as pltpu
from jax.experimental.pallas import tpu_sc as plsc

N = 10000
E = 320000
D = 128

NC = 2
NS = 16
NP = 10240
ROWS_PER_TILE = NP // NS
E_PER_TILE = E // (NC * NS)
CH = 80
NCHUNK = E_PER_TILE // CH
ZROWS = 40
NBUF = 3

_MESH = plsc.VectorSubcoreMesh(core_axis_name="c", subcore_axis_name="s")


def _propagate_body(xs_hbm, src_hbm, dst_hbm, agg_hbm, sidx_all, rows, dbuf,
                    zbuf, acc, sgs, sss, sds):
    c = lax.axis_index("c")
    s = lax.axis_index("s")

    zero16 = jnp.zeros((16,), jnp.float32)

    def fz(i, carry):
        for j in range(D // 16):
            zbuf[i, pl.ds(j * 16, 16)] = zero16
        return carry

    lax.fori_loop(0, ZROWS, fz, 0)

    def fcz(j, carry):
        pltpu.async_copy(
            zbuf, acc.at[pl.ds(s * ROWS_PER_TILE + j * ZROWS, ZROWS)], sss[0]
        )
        return carry

    lax.fori_loop(0, ROWS_PER_TILE // ZROWS, fcz, 0)

    def fcw(j, carry):
        pltpu.make_async_copy(
            zbuf, acc.at[pl.ds(s * ROWS_PER_TILE + j * ZROWS, ZROWS)], sss[0]
        ).wait()
        return carry

    lax.fori_loop(0, ROWS_PER_TILE // ZROWS, fcw, 0)

    tb = (c * NS + s) * E_PER_TILE
    pltpu.sync_copy(src_hbm.at[pl.ds(tb, E_PER_TILE)], sidx_all)
    plsc.subcore_barrier()

    def _prefetch(i, p):
        pltpu.async_copy(dst_hbm.at[pl.ds(tb + i * CH, CH)], dbuf[p], sds[p])
        pltpu.async_copy(
            xs_hbm.at[sidx_all.at[pl.ds(i * CH, CH)]], rows[p], sgs[p]
        )

    def _wait_pref(i, p):
        pltpu.make_async_copy(dst_hbm.at[pl.ds(tb, CH)], dbuf[p],
                              sds[p]).wait()
        pltpu.make_async_copy(
            xs_hbm.at[sidx_all.at[pl.ds(i * CH, CH)]], rows[p], sgs[p]
        ).wait()

    def _scatter(p):
        pltpu.async_copy(rows[p], acc.at[dbuf[p]], sss[p], add=True)

    def _wait_scatter(p):
        pltpu.make_async_copy(rows[p], acc.at[dbuf[p]], sss[p]).wait()

    for p in range(NBUF):
        _prefetch(p, p)

    NK = NCHUNK // NBUF

    def body(k, carry):
        i0 = k * NBUF
        for p in range(NBUF):
            _wait_pref(i0 + p, p)
            _scatter(p)

        @pl.when(k < NK - 1)
        def _():
            for p in range(NBUF):
                _wait_scatter(p)
                _prefetch(i0 + NBUF + p, p)

        return carry

    lax.fori_loop(0, NK, body, 0)
    for p in range(NBUF):
        _wait_scatter(p)

    for r in range(NCHUNK - NK * NBUF):
        i = NK * NBUF + r
        _prefetch(i, 0)
        _wait_pref(i, 0)
        _scatter(0)
        _wait_scatter(0)

    plsc.subcore_barrier()
    pltpu.sync_copy(
        acc.at[pl.ds(s * ROWS_PER_TILE, ROWS_PER_TILE)],
        agg_hbm.at[c].at[pl.ds(s * ROWS_PER_TILE, ROWS_PER_TILE)],
    )


_propagate_call = pl.kernel(
    _propagate_body,
    out_type=jax.ShapeDtypeStruct((NC, NP, D), jnp.float32),
    mesh=_MESH,
    scratch_types=[
        pltpu.VMEM((E_PER_TILE,), jnp.int32),
        [pltpu.VMEM((CH, D), jnp.float32) for _ in range(NBUF)],
        [pltpu.VMEM((CH,), jnp.int32) for _ in range(NBUF)],
        pltpu.VMEM((ZROWS, D), jnp.float32),
        pltpu.VMEM_SHARED((NP, D), jnp.float32),
        [pltpu.SemaphoreType.DMA for _ in range(NBUF)],
        [pltpu.SemaphoreType.DMA for _ in range(NBUF)],
        [pltpu.SemaphoreType.DMA for _ in range(NBUF)],
    ],
)


BN = 400


def _scales(dgo_blk, dgi_blk):
    deg_out = dgo_blk[0, :, 0:1] + dgo_blk[1, :, 0:1]
    deg_in = dgi_blk[0, :, 0:1] + dgi_blk[1, :, 0:1]
    s_out = lax.rsqrt(jnp.maximum(deg_out, 1.0))
    s_in = lax.rsqrt(jnp.maximum(deg_in, 1.0))
    return s_out, s_in


_deg_spec = pl.BlockSpec((NC, BN, 16), lambda i: (0, i, 0))


def _pre_body(x_ref, dgo_ref, dgi_ref, out_ref):
    s_out, _ = _scales(dgo_ref[...], dgi_ref[...])
    out_ref[...] = x_ref[...] * s_out


_pre_call = pl.pallas_call(
    _pre_body,
    grid=(N // BN,),
    in_specs=[
        pl.BlockSpec((BN, D), lambda i: (i, 0)),
        _deg_spec,
        _deg_spec,
    ],
    out_specs=pl.BlockSpec((BN, D), lambda i: (i, 0)),
    out_shape=jax.ShapeDtypeStruct((N, D), jnp.float32),
)


def _mid_body(agg_ref, dgo_ref, dgi_ref, w_ref, b_ref, out_ref):
    s_out, s_in = _scales(dgo_ref[...], dgi_ref[...])
    a = agg_ref[0, :, :] + agg_ref[1, :, :]
    y = jnp.dot(a * s_in, w_ref[...], preferred_element_type=jnp.float32)
    out_ref[...] = (y + b_ref[...]) * s_out


_mid_call = pl.pallas_call(
    _mid_body,
    grid=(N // BN,),
    in_specs=[
        pl.BlockSpec((NC, BN, D), lambda i: (0, i, 0)),
        _deg_spec,
        _deg_spec,
        pl.BlockSpec((D, D), lambda i: (0, 0)),
        pl.BlockSpec((1, D), lambda i: (0, 0)),
    ],
    out_specs=pl.BlockSpec((BN, D), lambda i: (i, 0)),
    out_shape=jax.ShapeDtypeStruct((N, D), jnp.float32),
)


def _final_body(agg_ref, dgo_ref, dgi_ref, w4_ref, b4_ref, wl_ref, bl_ref,
                out_ref):
    _, s_in = _scales(dgo_ref[...], dgi_ref[...])
    a = agg_ref[0, :, :] + agg_ref[1, :, :]
    y = jnp.dot(a * s_in, w4_ref[...], preferred_element_type=jnp.float32)
    y = y + b4_ref[...]
    z = jnp.dot(y, wl_ref[...], preferred_element_type=jnp.float32)
    out_ref[...] = jnp.maximum(z + bl_ref[...], 0.0)


_final_call = pl.pallas_call(
    _final_body,
    grid=(N // BN,),
    in_specs=[
        pl.BlockSpec((NC, BN, D), lambda i: (0, i, 0)),
        _deg_spec,
        _deg_spec,
        pl.BlockSpec((D, D), lambda i: (0, 0)),
        pl.BlockSpec((1, D), lambda i: (0, 0)),
        pl.BlockSpec((D, D), lambda i: (0, 0)),
        pl.BlockSpec((1, D), lambda i: (0, 0)),
    ],
    out_specs=pl.BlockSpec((BN, D), lambda i: (i, 0)),
    out_shape=jax.ShapeDtypeStruct((N, D), jnp.float32),
)


def kernel(in_feat, edge_index, W0, b0, W1, b1, W2, b2, W3, b3, W4, b4, Wl, bl):
    src = edge_index[0]
    dst = edge_index[1]
    ones_mat = jnp.ones((N, D), jnp.float32)

    dgo = _propagate_call(ones_mat, dst, src)[:, :, :16]
    dgi = _propagate_call(ones_mat + jnp.minimum(dgo[0, 0, 0], 0.0), src, dst)[:, :, :16]

    xs = _pre_call(in_feat, dgo, dgi)
    for W, b in zip([W0, W1, W2, W3], [b0, b1, b2, b3]):
        agg = _propagate_call(xs, src, dst)
        xs = _mid_call(agg, dgo, dgi, W, b.reshape(1, D))
    agg = _propagate_call(xs, src, dst)
    return _final_call(agg, dgo, dgi, W4, b4.reshape(1, D), Wl, bl.reshape(1, D))


# speedup vs baseline: 26.5735x; 1.0681x over previous
import jax
import jax.numpy as jnp
from jax import lax
from jax.experimental import pallas as pl
from jax.experimental.pallas import tpu as pltpu
from jax.experimental.pallas import tpu_sc as plsc

N = 10000
E = 320000
D = 128

NC = 2
NS = 16
NP = 10240
ROWS_PER_TILE = NP // NS
E_PER_TILE = E // (NC * NS)
CH = 40
NCHUNK = E_PER_TILE // CH
ZROWS = 40
NBUF = 5

_MESH = plsc.VectorSubcoreMesh(core_axis_name="c", subcore_axis_name="s")


def _propagate_body(xs_hbm, src_hbm, dst_hbm, agg_hbm, sidx_all, rows, dbuf,
                    zbuf, acc, sgs, sss, sds):
    c = lax.axis_index("c")
    s = lax.axis_index("s")

    zero16 = jnp.zeros((16,), jnp.float32)

    def fz(i, carry):
        for j in range(D // 16):
            zbuf[i, pl.ds(j * 16, 16)] = zero16
        return carry

    lax.fori_loop(0, ZROWS, fz, 0)

    def fcz(j, carry):
        pltpu.async_copy(
            zbuf, acc.at[pl.ds(s * ROWS_PER_TILE + j * ZROWS, ZROWS)], sss[0]
        )
        return carry

    lax.fori_loop(0, ROWS_PER_TILE // ZROWS, fcz, 0)

    def fcw(j, carry):
        pltpu.make_async_copy(
            zbuf, acc.at[pl.ds(s * ROWS_PER_TILE + j * ZROWS, ZROWS)], sss[0]
        ).wait()
        return carry

    lax.fori_loop(0, ROWS_PER_TILE // ZROWS, fcw, 0)

    tb = (c * NS + s) * E_PER_TILE
    pltpu.sync_copy(src_hbm.at[pl.ds(tb, E_PER_TILE)], sidx_all)
    plsc.subcore_barrier()

    def _prefetch(i, p):
        pltpu.async_copy(dst_hbm.at[pl.ds(tb + i * CH, CH)], dbuf[p], sds[p])
        pltpu.async_copy(
            xs_hbm.at[sidx_all.at[pl.ds(i * CH, CH)]], rows[p], sgs[p]
        )

    def _wait_pref(i, p):
        pltpu.make_async_copy(dst_hbm.at[pl.ds(tb, CH)], dbuf[p],
                              sds[p]).wait()
        pltpu.make_async_copy(
            xs_hbm.at[sidx_all.at[pl.ds(i * CH, CH)]], rows[p], sgs[p]
        ).wait()

    def _scatter(p):
        pltpu.async_copy(rows[p], acc.at[dbuf[p]], sss[p], add=True)

    def _wait_scatter(p):
        pltpu.make_async_copy(rows[p], acc.at[dbuf[p]], sss[p]).wait()

    for p in range(NBUF):
        _prefetch(p, p)

    NK = NCHUNK // NBUF

    def body(k, carry):
        i0 = k * NBUF
        for p in range(NBUF):
            _wait_pref(i0 + p, p)
            _scatter(p)

        @pl.when(k < NK - 1)
        def _():
            for p in range(NBUF):
                _wait_scatter(p)
                _prefetch(i0 + NBUF + p, p)

        return carry

    lax.fori_loop(0, NK, body, 0)
    for p in range(NBUF):
        _wait_scatter(p)

    for r in range(NCHUNK - NK * NBUF):
        i = NK * NBUF + r
        _prefetch(i, 0)
        _wait_pref(i, 0)
        _scatter(0)
        _wait_scatter(0)

    plsc.subcore_barrier()
    pltpu.sync_copy(
        acc.at[pl.ds(s * ROWS_PER_TILE, ROWS_PER_TILE)],
        agg_hbm.at[c].at[pl.ds(s * ROWS_PER_TILE, ROWS_PER_TILE)],
    )


_propagate_call = pl.kernel(
    _propagate_body,
    out_type=jax.ShapeDtypeStruct((NC, NP, D), jnp.float32),
    mesh=_MESH,
    scratch_types=[
        pltpu.VMEM((E_PER_TILE,), jnp.int32),
        [pltpu.VMEM((CH, D), jnp.float32) for _ in range(NBUF)],
        [pltpu.VMEM((CH,), jnp.int32) for _ in range(NBUF)],
        pltpu.VMEM((ZROWS, D), jnp.float32),
        pltpu.VMEM_SHARED((NP, D), jnp.float32),
        [pltpu.SemaphoreType.DMA for _ in range(NBUF)],
        [pltpu.SemaphoreType.DMA for _ in range(NBUF)],
        [pltpu.SemaphoreType.DMA for _ in range(NBUF)],
    ],
)


BN = 400


def _scales(dgo_blk, dgi_blk):
    deg_out = dgo_blk[0, :, 0:1] + dgo_blk[1, :, 0:1]
    deg_in = dgi_blk[0, :, 0:1] + dgi_blk[1, :, 0:1]
    s_out = lax.rsqrt(jnp.maximum(deg_out, 1.0))
    s_in = lax.rsqrt(jnp.maximum(deg_in, 1.0))
    return s_out, s_in


_deg_spec = pl.BlockSpec((NC, BN, 16), lambda i: (0, i, 0))


def _pre_body(x_ref, dgo_ref, dgi_ref, out_ref):
    s_out, _ = _scales(dgo_ref[...], dgi_ref[...])
    out_ref[...] = x_ref[...] * s_out


_pre_call = pl.pallas_call(
    _pre_body,
    grid=(N // BN,),
    in_specs=[
        pl.BlockSpec((BN, D), lambda i: (i, 0)),
        _deg_spec,
        _deg_spec,
    ],
    out_specs=pl.BlockSpec((BN, D), lambda i: (i, 0)),
    out_shape=jax.ShapeDtypeStruct((N, D), jnp.float32),
)


def _mid_body(agg_ref, dgo_ref, dgi_ref, w_ref, b_ref, out_ref):
    s_out, s_in = _scales(dgo_ref[...], dgi_ref[...])
    a = agg_ref[0, :, :] + agg_ref[1, :, :]
    y = jnp.dot(a * s_in, w_ref[...], preferred_element_type=jnp.float32)
    out_ref[...] = (y + b_ref[...]) * s_out


_mid_call = pl.pallas_call(
    _mid_body,
    grid=(N // BN,),
    in_specs=[
        pl.BlockSpec((NC, BN, D), lambda i: (0, i, 0)),
        _deg_spec,
        _deg_spec,
        pl.BlockSpec((D, D), lambda i: (0, 0)),
        pl.BlockSpec((1, D), lambda i: (0, 0)),
    ],
    out_specs=pl.BlockSpec((BN, D), lambda i: (i, 0)),
    out_shape=jax.ShapeDtypeStruct((N, D), jnp.float32),
)


def _final_body(agg_ref, dgo_ref, dgi_ref, w4_ref, b4_ref, wl_ref, bl_ref,
                out_ref):
    _, s_in = _scales(dgo_ref[...], dgi_ref[...])
    a = agg_ref[0, :, :] + agg_ref[1, :, :]
    y = jnp.dot(a * s_in, w4_ref[...], preferred_element_type=jnp.float32)
    y = y + b4_ref[...]
    z = jnp.dot(y, wl_ref[...], preferred_element_type=jnp.float32)
    out_ref[...] = jnp.maximum(z + bl_ref[...], 0.0)


_final_call = pl.pallas_call(
    _final_body,
    grid=(N // BN,),
    in_specs=[
        pl.BlockSpec((NC, BN, D), lambda i: (0, i, 0)),
        _deg_spec,
        _deg_spec,
        pl.BlockSpec((D, D), lambda i: (0, 0)),
        pl.BlockSpec((1, D), lambda i: (0, 0)),
        pl.BlockSpec((D, D), lambda i: (0, 0)),
        pl.BlockSpec((1, D), lambda i: (0, 0)),
    ],
    out_specs=pl.BlockSpec((BN, D), lambda i: (i, 0)),
    out_shape=jax.ShapeDtypeStruct((N, D), jnp.float32),
)


def kernel(in_feat, edge_index, W0, b0, W1, b1, W2, b2, W3, b3, W4, b4, Wl, bl):
    src = edge_index[0]
    dst = edge_index[1]
    ones_mat = jnp.ones((N, D), jnp.float32)

    dgo = _propagate_call(ones_mat, dst, src)[:, :, :16]
    dgi = _propagate_call(ones_mat + jnp.minimum(dgo[0, 0, 0], 0.0), src, dst)[:, :, :16]

    xs = _pre_call(in_feat, dgo, dgi)
    for W, b in zip([W0, W1, W2, W3], [b0, b1, b2, b3]):
        agg = _propagate_call(xs, src, dst)
        xs = _mid_call(agg, dgo, dgi, W, b.reshape(1, D))
    agg = _propagate_call(xs, src, dst)
    return _final_call(agg, dgo, dgi, W4, b4.reshape(1, D), Wl, bl.reshape(1, D))
